# Initial kernel scaffold; baseline (speedup 1.0000x reference)
import jax
import jax.numpy as jnp
from jax.experimental import pallas as pl


def kernel(emission_factors, start_factors, stop_factors, transition_factors):
    raise NotImplementedError("write your pallas kernel here")


# speedup vs baseline: 6.5821x; 6.5821x over previous
import jax
import jax.numpy as jnp
from jax.experimental import pallas as pl
from jax.experimental.pallas import tpu as pltpu

_T, _B, _S = 4096, 16, 64
_CHUNK = 512
_NBLK = _T // _CHUNK


def _fwd_body(em_ref, start_ref, stop_ref, trans_ref, out_ref, alphas_ref):
    t = pl.program_id(0)
    trans = trans_ref[...]
    tmax = jnp.max(trans)
    e2 = jnp.exp(trans - tmax)

    def step(s, alphas):
        m = jnp.max(alphas, axis=1, keepdims=True)
        p = jnp.exp(alphas - m)
        acc = jax.lax.dot_general(
            p, e2, (((1,), (1,)), ((), ())), preferred_element_type=jnp.float32
        )
        return (m + tmax) + (jnp.log(acc) + em_ref[s])

    @pl.when(t == 0)
    def _():
        a0 = start_ref[0, :][None, :] + em_ref[0]
        alphas_ref[...] = jax.lax.fori_loop(1, _CHUNK, step, a0, unroll=8)

    @pl.when(t > 0)
    def _():
        alphas_ref[...] = jax.lax.fori_loop(
            0, _CHUNK, step, alphas_ref[...], unroll=8
        )

    @pl.when(t == _NBLK - 1)
    def _():
        a = alphas_ref[...] + stop_ref[0, :][None, :]
        m = jnp.max(a, axis=1, keepdims=True)
        lse = m + jnp.log(jnp.sum(jnp.exp(a - m), axis=1, keepdims=True))
        out_ref[...] = lse.T


def kernel(emission_factors, start_factors, stop_factors, transition_factors):
    out = pl.pallas_call(
        _fwd_body,
        grid=(_NBLK,),
        in_specs=[
            pl.BlockSpec((_CHUNK, _B, _S), lambda i: (i, 0, 0)),
            pl.BlockSpec((1, _S), lambda i: (0, 0)),
            pl.BlockSpec((1, _S), lambda i: (0, 0)),
            pl.BlockSpec((_S, _S), lambda i: (0, 0)),
        ],
        out_specs=pl.BlockSpec((1, _B), lambda i: (0, 0)),
        out_shape=jax.ShapeDtypeStruct((1, _B), jnp.float32),
        scratch_shapes=[pltpu.VMEM((_B, _S), jnp.float32)],
    )(
        emission_factors,
        start_factors.reshape(1, _S),
        stop_factors.reshape(1, _S),
        transition_factors,
    )
    return out.reshape(_B)


# speedup vs baseline: 10.4650x; 1.5899x over previous
import jax
import jax.numpy as jnp
from jax.experimental import pallas as pl
from jax.experimental.pallas import tpu as pltpu

_T, _B, _S = 4096, 16, 64
_CHUNK = 512
_NBLK = _T // _CHUNK
_LN2 = 0.6931471805599453
_EXP_MASK = 0x7F800000
_EXP_2X_BIAS = 254 << 23


def _fwd_body(em_ref, start_ref, stop_ref, trans_ref, out_ref, p_ref, sc_ref):
    t = pl.program_id(0)
    trans = trans_ref[...]
    tmax = jnp.max(trans)
    e2 = jnp.exp(trans - tmax)

    def step(s, carry):
        p, scale, sc = carry
        comb = jnp.exp(em_ref[s]) * scale
        q = jax.lax.dot_general(
            p, e2, (((1,), (1,)), ((), ())), preferred_element_type=jnp.float32
        )
        p2 = q * comb
        c = jnp.max(p2, axis=1, keepdims=True)
        eb = jax.lax.bitcast_convert_type(c, jnp.int32) & _EXP_MASK
        scale2 = jax.lax.bitcast_convert_type(_EXP_2X_BIAS - eb, jnp.float32)
        sc2 = sc + jax.lax.shift_right_arithmetic(eb, 23) - 127
        return (p2, scale2, sc2)

    def run(lo, p0, sc0):
        ones = jnp.ones((_B, 1), jnp.float32)
        p, scale, sc = jax.lax.fori_loop(
            lo, _CHUNK, step, (p0, ones, sc0), unroll=8
        )
        p_ref[...] = p * scale
        sc_ref[...] = sc

    @pl.when(t == 0)
    def _():
        a0 = start_ref[0, :][None, :] + em_ref[0]
        run(1, jnp.exp(a0), jnp.zeros((_B, 1), jnp.int32))

    @pl.when(t > 0)
    def _():
        run(0, p_ref[...], sc_ref[...])

    @pl.when(t == _NBLK - 1)
    def _():
        stop = stop_ref[0, :]
        smax = jnp.max(stop)
        w = p_ref[...] * jnp.exp(stop - smax)[None, :]
        lse = jnp.log(jnp.sum(w, axis=1, keepdims=True))
        off = sc_ref[...].astype(jnp.float32) * _LN2 + (_T - 1) * tmax + smax
        out_ref[...] = (off + lse).T


def kernel(emission_factors, start_factors, stop_factors, transition_factors):
    out = pl.pallas_call(
        _fwd_body,
        grid=(_NBLK,),
        in_specs=[
            pl.BlockSpec((_CHUNK, _B, _S), lambda i: (i, 0, 0)),
            pl.BlockSpec((1, _S), lambda i: (0, 0)),
            pl.BlockSpec((1, _S), lambda i: (0, 0)),
            pl.BlockSpec((_S, _S), lambda i: (0, 0)),
        ],
        out_specs=pl.BlockSpec((1, _B), lambda i: (0, 0)),
        out_shape=jax.ShapeDtypeStruct((1, _B), jnp.float32),
        scratch_shapes=[
            pltpu.VMEM((_B, _S), jnp.float32),
            pltpu.VMEM((_B, 1), jnp.int32),
        ],
    )(
        emission_factors,
        start_factors.reshape(1, _S),
        stop_factors.reshape(1, _S),
        transition_factors,
    )
    return out.reshape(_B)


# speedup vs baseline: 96.0279x; 9.1761x over previous
import jax
import jax.numpy as jnp
from jax.experimental import pallas as pl
from jax.experimental.pallas import tpu as pltpu

_T, _B, _S = 4096, 16, 64
_K = 32
_C = _T // _K
_SUB = 16
_NSUB = _C // _SUB
_R = _K * _B
_LN2 = 0.6931471805599453
_EXP_MASK = 0x7F800000


def _fwd_body(emf_ref, emb_ref, start_ref, stop_ref, trans_ref, out_ref,
              f_ref, b_ref, scf_ref):
    i = pl.program_id(0)
    trans = trans_ref[...]
    tmax = jnp.max(trans)
    e2 = jnp.exp(trans - tmax)

    def pos_fwd(F, comb):
        q = jax.lax.dot_general(
            F, e2, (((1,), (1,)), ((), ())), preferred_element_type=jnp.float32
        )
        return q * comb

    def pos_bwd(Bw, comb):
        return jax.lax.dot_general(
            Bw * comb, e2, (((1,), (0,)), ((), ())),
            preferred_element_type=jnp.float32,
        )

    def renorm_scale(x):
        c = jnp.max(x, axis=1, keepdims=True)
        eb = jax.lax.bitcast_convert_type(c, jnp.int32) & _EXP_MASK
        scale = jax.lax.bitcast_convert_type((254 << 23) - eb, jnp.float32)
        return scale, eb

    def sweep(F, Bw, sf, scf, sb, start_pos):
        for s in range(start_pos, _SUB):
            comb_f = jnp.exp(emf_ref[:, s].reshape(_R, _S))
            comb_b = jnp.exp(emb_ref[:, _SUB - 1 - s].reshape(_R, _S))
            if sf is not None:
                comb_f = comb_f * sf
                comb_b = comb_b * sb
                sf = sb = None
            F = pos_fwd(F, comb_f)
            Bw = pos_bwd(Bw, comb_b)
            if s % 4 == 3:
                sf, ebf = renorm_scale(F)
                sb, _ = renorm_scale(Bw)
                scf = scf + jax.lax.shift_right_arithmetic(ebf, 23) - 127
        f_ref[...] = F * sf
        b_ref[...] = Bw * sb
        scf_ref[...] = scf

    @pl.when(i == 0)
    def _():
        ones = jnp.ones((_R, _S), jnp.float32)
        F = pos_fwd(ones, jnp.exp(emf_ref[:, 0].reshape(_R, _S)))
        Bw = pos_bwd(ones, jnp.exp(emb_ref[:, _SUB - 1].reshape(_R, _S)))
        p0 = jnp.exp(start_ref[0, :][None, :] + emf_ref[0, 0])
        row = jax.lax.broadcasted_iota(jnp.int32, (_R, _S), 0)
        F = jnp.where(row < _B, jnp.concatenate([p0] * _K, axis=0), F)
        sweep(F, Bw, None, jnp.zeros((_R, 1), jnp.int32), None, 1)

    @pl.when(i > 0)
    def _():
        sweep(f_ref[...], b_ref[...], None, scf_ref[...], None, 0)

    @pl.when(i == _NSUB - 1)
    def _():
        F = f_ref[...]
        Bw = b_ref[...]
        scf = scf_ref[...]
        dk = jnp.sum(F[: _R - _B] * Bw[_B:], axis=1, keepdims=True)
        sk = jnp.sum(Bw[_B:], axis=1, keepdims=True)
        V = jnp.log(dk) - jnp.log(sk)
        stop = stop_ref[0, :]
        smax = jnp.max(stop)
        w = jnp.sum(
            F[_R - _B:] * jnp.exp(stop - smax)[None, :], axis=1, keepdims=True
        )
        acc = jnp.log(w)
        for j in range(_K - 1):
            acc = acc + V[j * _B : (j + 1) * _B]
        scft = scf[: _B]
        for j in range(1, _K):
            scft = scft + scf[j * _B : (j + 1) * _B]
        out = acc + scft.astype(jnp.float32) * _LN2 + smax + (_T - 1) * tmax
        out_ref[...] = out.T


def kernel(emission_factors, start_factors, stop_factors, transition_factors):
    em4 = emission_factors.reshape(_K, _C, _B, _S)
    out = pl.pallas_call(
        _fwd_body,
        grid=(_NSUB,),
        in_specs=[
            pl.BlockSpec((_K, _SUB, _B, _S), lambda i: (0, i, 0, 0)),
            pl.BlockSpec((_K, _SUB, _B, _S), lambda i: (0, _NSUB - 1 - i, 0, 0)),
            pl.BlockSpec((1, _S), lambda i: (0, 0)),
            pl.BlockSpec((1, _S), lambda i: (0, 0)),
            pl.BlockSpec((_S, _S), lambda i: (0, 0)),
        ],
        out_specs=pl.BlockSpec((1, _B), lambda i: (0, 0)),
        out_shape=jax.ShapeDtypeStruct((1, _B), jnp.float32),
        scratch_shapes=[
            pltpu.VMEM((_R, _S), jnp.float32),
            pltpu.VMEM((_R, _S), jnp.float32),
            pltpu.VMEM((_R, 1), jnp.int32),
        ],
    )(
        em4,
        em4,
        start_factors.reshape(1, _S),
        stop_factors.reshape(1, _S),
        transition_factors,
    )
    return out.reshape(_B)


# speedup vs baseline: 101.4716x; 1.0567x over previous
import jax
import jax.numpy as jnp
from jax.experimental import pallas as pl
from jax.experimental.pallas import tpu as pltpu

_T, _B, _S = 4096, 16, 64
_K = 32
_C = _T // _K
_SUB = 16
_NSUB = _C // _SUB
_R = _K * _B
_LN2 = 0.6931471805599453
_EXP_MASK = 0x7F800000


def _fwd_body(em_hbm, start_ref, stop_ref, trans_ref, out_ref,
              f_ref, b_ref, scf_ref, fbuf, bbuf, dsem):
    i = pl.program_id(0)
    slot = jax.lax.rem(i, 2)
    trans = trans_ref[...]
    tmax = jnp.max(trans)
    e2 = jnp.exp(trans - tmax)

    def fcopy(sl, step, k):
        return pltpu.make_async_copy(
            em_hbm.at[pl.ds(k * _C + step * _SUB, _SUB)],
            fbuf.at[sl, k],
            dsem.at[0, sl],
        )

    def bcopy(sl, step, k):
        return pltpu.make_async_copy(
            em_hbm.at[pl.ds(k * _C + (_NSUB - 1 - step) * _SUB, _SUB)],
            bbuf.at[sl, k],
            dsem.at[1, sl],
        )

    def issue(sl, step):
        for k in range(_K):
            fcopy(sl, step, k).start()
            bcopy(sl, step, k).start()

    def wait(sl, step):
        for k in range(_K):
            fcopy(sl, step, k).wait()
            bcopy(sl, step, k).wait()

    @pl.when(i == 0)
    def _():
        issue(0, 0)

    @pl.when(i + 1 < _NSUB)
    def _():
        issue(jax.lax.rem(i + 1, 2), i + 1)

    wait(slot, i)

    def pos_fwd(F, comb):
        q = jax.lax.dot_general(
            F, e2, (((1,), (1,)), ((), ())), preferred_element_type=jnp.float32
        )
        return q * comb

    def pos_bwd(Bw, comb):
        return jax.lax.dot_general(
            Bw * comb, e2, (((1,), (0,)), ((), ())),
            preferred_element_type=jnp.float32,
        )

    def renorm_scale(x):
        c = jnp.max(x, axis=1, keepdims=True)
        eb = jax.lax.bitcast_convert_type(c, jnp.int32) & _EXP_MASK
        scale = jax.lax.bitcast_convert_type((254 << 23) - eb, jnp.float32)
        return scale, eb

    def emf_at(s):
        return fbuf[slot, :, s].reshape(_R, _S)

    def emb_at(s):
        return bbuf[slot, :, _SUB - 1 - s].reshape(_R, _S)

    def sweep(F, Bw, sf, scf, sb, start_pos):
        for s in range(start_pos, _SUB):
            comb_f = jnp.exp(emf_at(s))
            comb_b = jnp.exp(emb_at(s))
            if sf is not None:
                comb_f = comb_f * sf
                comb_b = comb_b * sb
                sf = sb = None
            F = pos_fwd(F, comb_f)
            Bw = pos_bwd(Bw, comb_b)
            if s % 4 == 3:
                sf, ebf = renorm_scale(F)
                sb, _ = renorm_scale(Bw)
                scf = scf + jax.lax.shift_right_arithmetic(ebf, 23) - 127
        f_ref[...] = F * sf
        b_ref[...] = Bw * sb
        scf_ref[...] = scf

    @pl.when(i == 0)
    def _():
        ones = jnp.ones((_R, _S), jnp.float32)
        F = pos_fwd(ones, jnp.exp(emf_at(0)))
        Bw = pos_bwd(ones, jnp.exp(emb_at(0)))
        p0 = jnp.exp(start_ref[0, :][None, :] + fbuf[slot, 0, 0])
        row = jax.lax.broadcasted_iota(jnp.int32, (_R, _S), 0)
        F = jnp.where(row < _B, jnp.concatenate([p0] * _K, axis=0), F)
        sweep(F, Bw, None, jnp.zeros((_R, 1), jnp.int32), None, 1)

    @pl.when(i > 0)
    def _():
        sweep(f_ref[...], b_ref[...], None, scf_ref[...], None, 0)

    @pl.when(i == _NSUB - 1)
    def _():
        F = f_ref[...]
        Bw = b_ref[...]
        scf = scf_ref[...]
        dk = jnp.sum(F[: _R - _B] * Bw[_B:], axis=1, keepdims=True)
        sk = jnp.sum(Bw[_B:], axis=1, keepdims=True)
        V = jnp.log(dk) - jnp.log(sk)
        stop = stop_ref[0, :]
        smax = jnp.max(stop)
        w = jnp.sum(
            F[_R - _B:] * jnp.exp(stop - smax)[None, :], axis=1, keepdims=True
        )
        acc = jnp.log(w)
        for j in range(_K - 1):
            acc = acc + V[j * _B : (j + 1) * _B]
        scft = scf[: _B]
        for j in range(1, _K):
            scft = scft + scf[j * _B : (j + 1) * _B]
        out = acc + scft.astype(jnp.float32) * _LN2 + smax + (_T - 1) * tmax
        out_ref[...] = out.T


def kernel(emission_factors, start_factors, stop_factors, transition_factors):
    out = pl.pallas_call(
        _fwd_body,
        grid=(_NSUB,),
        in_specs=[
            pl.BlockSpec(memory_space=pl.ANY),
            pl.BlockSpec((1, _S), lambda i: (0, 0)),
            pl.BlockSpec((1, _S), lambda i: (0, 0)),
            pl.BlockSpec((_S, _S), lambda i: (0, 0)),
        ],
        out_specs=pl.BlockSpec((1, _B), lambda i: (0, 0)),
        out_shape=jax.ShapeDtypeStruct((1, _B), jnp.float32),
        scratch_shapes=[
            pltpu.VMEM((_R, _S), jnp.float32),
            pltpu.VMEM((_R, _S), jnp.float32),
            pltpu.VMEM((_R, 1), jnp.int32),
            pltpu.VMEM((2, _K, _SUB, _B, _S), jnp.float32),
            pltpu.VMEM((2, _K, _SUB, _B, _S), jnp.float32),
            pltpu.SemaphoreType.DMA((2, 2)),
        ],
    )(
        emission_factors,
        start_factors.reshape(1, _S),
        stop_factors.reshape(1, _S),
        transition_factors,
    )
    return out.reshape(_B)
